# baseline (speedup 1.0000x reference)
import functools

import jax
import jax.numpy as jnp
from jax import lax
from jax.experimental import pallas as pl
from jax.experimental.pallas import tpu as pltpu
from jax.experimental.pallas import tpu_sc as plsc

_NUM_CORES = 2
_NUM_SUBCORES = 16
_NUM_WORKERS = _NUM_CORES * _NUM_SUBCORES
_LANES = 16
_CHUNK = 16


@functools.partial(jax.jit, static_argnums=(3, 4))
def _sc_add_pe(emb2d, idx, pe, n_rows, dim):
    rows_per_w = n_rows // _NUM_WORKERS
    steps = rows_per_w // _CHUNK
    mesh = plsc.VectorSubcoreMesh(core_axis_name="c", subcore_axis_name="s")

    @functools.partial(
        pl.kernel,
        mesh=mesh,
        out_type=jax.ShapeDtypeStruct((n_rows, dim), jnp.float32),
        scratch_types=[
            pltpu.VMEM((rows_per_w,), jnp.int32),
            pltpu.VMEM((_CHUNK, dim), jnp.float32),
            pltpu.VMEM((_CHUNK, dim), jnp.float32),
            pltpu.SemaphoreType.DMA,
            pltpu.SemaphoreType.DMA,
        ],
    )
    def k(emb_hbm, idx_hbm, pe_hbm, out_hbm, idx_v, pe_buf, emb_buf, sem_g, sem_e):
        wid = lax.axis_index("s") * _NUM_CORES + lax.axis_index("c")
        wbase = wid * rows_per_w
        pltpu.sync_copy(idx_hbm.at[pl.ds(wbase, rows_per_w)], idx_v)

        @pl.loop(0, steps)
        def _(step):
            base = wbase + step * _CHUNK
            cp_g = pltpu.async_copy(
                pe_hbm.at[idx_v.at[pl.ds(step * _CHUNK, _CHUNK)]], pe_buf, sem_g
            )
            cp_e = pltpu.async_copy(emb_hbm.at[pl.ds(base, _CHUNK)], emb_buf, sem_e)
            cp_g.wait()
            cp_e.wait()

            @pl.loop(0, _CHUNK)
            def _(r):
                for c in range(0, dim, _LANES):
                    plsc.addupdate(
                        emb_buf.at[r, pl.ds(c, _LANES)], pe_buf[r, pl.ds(c, _LANES)]
                    )

            pltpu.sync_copy(emb_buf, out_hbm.at[pl.ds(base, _CHUNK)])

    return k(emb2d, idx, pe)


def kernel(emb, shift, pe):
    bsz, length, dim = emb.shape
    mid_pos = pe.shape[0] // 2
    idx = (mid_pos + jnp.arange(length, dtype=jnp.int32))[None, :] - shift.astype(
        jnp.int32
    )[:, None]
    out = _sc_add_pe(
        emb.reshape(bsz * length, dim),
        idx.reshape(bsz * length),
        pe,
        bsz * length,
        dim,
    )
    return out.reshape(bsz, length, dim)


# speedup vs baseline: 1.5837x; 1.5837x over previous
import functools

import jax
import jax.numpy as jnp
from jax import lax
from jax.experimental import pallas as pl
from jax.experimental.pallas import tpu as pltpu
from jax.experimental.pallas import tpu_sc as plsc

_NUM_CORES = 2
_NUM_SUBCORES = 16
_NUM_WORKERS = _NUM_CORES * _NUM_SUBCORES
_LANES = 16
_CHUNK = 8
_NBUF = 4


@functools.partial(jax.jit, static_argnums=(3, 4))
def _sc_add_pe(emb2d, idx, pe, n_rows, dim):
    rows_per_w = n_rows // _NUM_WORKERS
    steps = rows_per_w // _CHUNK
    assert steps % _NBUF == 0 and steps >= 2 * _NBUF
    mesh = plsc.VectorSubcoreMesh(core_axis_name="c", subcore_axis_name="s")

    scratch = (
        [pltpu.VMEM((rows_per_w,), jnp.int32)]
        + [pltpu.VMEM((_CHUNK, dim), jnp.float32)] * (2 * _NBUF)
        + [pltpu.SemaphoreType.DMA] * (3 * _NBUF)
    )

    @functools.partial(
        pl.kernel,
        mesh=mesh,
        out_type=jax.ShapeDtypeStruct((n_rows, dim), jnp.float32),
        scratch_types=scratch,
    )
    def k(emb_hbm, idx_hbm, pe_hbm, out_hbm, idx_v, *bufs_and_sems):
        pe_bufs = bufs_and_sems[:_NBUF]
        emb_bufs = bufs_and_sems[_NBUF : 2 * _NBUF]
        sem_g = bufs_and_sems[2 * _NBUF : 3 * _NBUF]
        sem_e = bufs_and_sems[3 * _NBUF : 4 * _NBUF]
        sem_o = bufs_and_sems[4 * _NBUF : 5 * _NBUF]

        wid = lax.axis_index("s") * _NUM_CORES + lax.axis_index("c")
        wbase = wid * rows_per_w
        pltpu.sync_copy(idx_hbm.at[pl.ds(wbase, rows_per_w)], idx_v)

        def gather_in(kk, s):
            return pltpu.make_async_copy(
                pe_hbm.at[idx_v.at[pl.ds(kk * _CHUNK, _CHUNK)]], pe_bufs[s], sem_g[s]
            )

        def emb_in(kk, s):
            return pltpu.make_async_copy(
                emb_hbm.at[pl.ds(wbase + kk * _CHUNK, _CHUNK)], emb_bufs[s], sem_e[s]
            )

        def out_cp(kk, s):
            return pltpu.make_async_copy(
                pe_bufs[s], out_hbm.at[pl.ds(wbase + kk * _CHUNK, _CHUNK)], sem_o[s]
            )

        for b in range(2):
            gather_in(b, b).start()
            emb_in(b, b).start()

        @pl.loop(0, steps, step=_NBUF)
        def _(g):
            for b in range(_NBUF):
                kk = g + b
                s = b
                ps = (b + 2) % _NBUF
                gather_in(kk, s).wait()
                emb_in(kk, s).wait()

                @pl.loop(0, _CHUNK)
                def _(r):
                    for c in range(0, dim, _LANES):
                        plsc.addupdate(
                            pe_bufs[s].at[r, pl.ds(c, _LANES)],
                            emb_bufs[s][r, pl.ds(c, _LANES)],
                        )

                out_cp(kk, s).start()

                @pl.when(kk >= 2)
                def _():
                    out_cp(kk - 2, ps).wait()

                @pl.when(kk + 2 < steps)
                def _():
                    gather_in(kk + 2, ps).start()
                    emb_in(kk + 2, ps).start()

        for kk in (steps - 2, steps - 1):
            out_cp(kk, kk % _NBUF).wait()

    return k(emb2d, idx, pe)


def kernel(emb, shift, pe):
    bsz, length, dim = emb.shape
    mid_pos = pe.shape[0] // 2
    idx = (mid_pos + jnp.arange(length, dtype=jnp.int32))[None, :] - shift.astype(
        jnp.int32
    )[:, None]
    out = _sc_add_pe(
        emb.reshape(bsz * length, dim),
        idx.reshape(bsz * length),
        pe,
        bsz * length,
        dim,
    )
    return out.reshape(bsz, length, dim)


# speedup vs baseline: 1.6477x; 1.0404x over previous
import functools

import jax
import jax.numpy as jnp
from jax import lax
from jax.experimental import pallas as pl
from jax.experimental.pallas import tpu as pltpu
from jax.experimental.pallas import tpu_sc as plsc

_NUM_CORES = 2
_NUM_SUBCORES = 16
_NUM_WORKERS = _NUM_CORES * _NUM_SUBCORES
_LANES = 16
_CHUNK = 16
_NBUF = 4
_EBUF = 2


@functools.partial(jax.jit, static_argnums=(3, 4))
def _sc_add_pe(emb2d, idx, pe, n_rows, dim):
    rows_per_w = n_rows // _NUM_WORKERS
    steps = rows_per_w // _CHUNK
    assert steps % _NBUF == 0 and steps >= 2 * _NBUF
    mesh = plsc.VectorSubcoreMesh(core_axis_name="c", subcore_axis_name="s")

    scratch = (
        [pltpu.VMEM((rows_per_w,), jnp.int32)]
        + [pltpu.VMEM((_CHUNK, dim), jnp.float32)] * (_NBUF + _EBUF)
        + [pltpu.SemaphoreType.DMA] * (2 * _NBUF + _EBUF)
    )

    @functools.partial(
        pl.kernel,
        mesh=mesh,
        out_type=jax.ShapeDtypeStruct((n_rows, dim), jnp.float32),
        scratch_types=scratch,
    )
    def k(emb_hbm, idx_hbm, pe_hbm, out_hbm, idx_v, *bufs_and_sems):
        pe_bufs = bufs_and_sems[:_NBUF]
        emb_bufs = bufs_and_sems[_NBUF : _NBUF + _EBUF]
        rest = bufs_and_sems[_NBUF + _EBUF :]
        sem_g = rest[:_NBUF]
        sem_e = rest[_NBUF : _NBUF + _EBUF]
        sem_o = rest[_NBUF + _EBUF :]

        wid = lax.axis_index("s") * _NUM_CORES + lax.axis_index("c")
        wbase = wid * rows_per_w
        pltpu.sync_copy(idx_hbm.at[pl.ds(wbase, rows_per_w)], idx_v)

        def gather_in(kk, s):
            return pltpu.make_async_copy(
                pe_hbm.at[idx_v.at[pl.ds(kk * _CHUNK, _CHUNK)]], pe_bufs[s], sem_g[s]
            )

        def emb_in(kk, s):
            return pltpu.make_async_copy(
                emb_hbm.at[pl.ds(wbase + kk * _CHUNK, _CHUNK)], emb_bufs[s], sem_e[s]
            )

        def out_cp(kk, s):
            return pltpu.make_async_copy(
                pe_bufs[s], out_hbm.at[pl.ds(wbase + kk * _CHUNK, _CHUNK)], sem_o[s]
            )

        for b in range(2):
            gather_in(b, b).start()
            emb_in(b, b % _EBUF).start()

        @pl.loop(0, steps, step=_NBUF)
        def _(g):
            for b in range(_NBUF):
                kk = g + b
                s = b
                es = b % _EBUF
                ps = (b + 2) % _NBUF
                gather_in(kk, s).wait()
                emb_in(kk, es).wait()

                @pl.loop(0, _CHUNK)
                def _(r):
                    for c in range(0, dim, _LANES):
                        plsc.addupdate(
                            pe_bufs[s].at[r, pl.ds(c, _LANES)],
                            emb_bufs[es][r, pl.ds(c, _LANES)],
                        )

                out_cp(kk, s).start()

                @pl.when(kk >= 2)
                def _():
                    out_cp(kk - 2, ps).wait()

                @pl.when(kk + 2 < steps)
                def _():
                    gather_in(kk + 2, ps).start()
                    emb_in(kk + 2, es).start()

        for kk in (steps - 2, steps - 1):
            out_cp(kk, kk % _NBUF).wait()

    return k(emb2d, idx, pe)


def kernel(emb, shift, pe):
    bsz, length, dim = emb.shape
    mid_pos = pe.shape[0] // 2
    idx = (mid_pos + jnp.arange(length, dtype=jnp.int32))[None, :] - shift.astype(
        jnp.int32
    )[:, None]
    out = _sc_add_pe(
        emb.reshape(bsz * length, dim),
        idx.reshape(bsz * length),
        pe,
        bsz * length,
        dim,
    )
    return out.reshape(bsz, length, dim)
